# Initial kernel scaffold; baseline (speedup 1.0000x reference)
import jax
import jax.numpy as jnp
from jax.experimental import pallas as pl


def kernel(positions, weight):
    raise NotImplementedError("write your pallas kernel here")


# speedup vs baseline: 1.5308x; 1.5308x over previous
import functools

import jax
import jax.numpy as jnp
from jax import lax
from jax.experimental import pallas as pl
from jax.experimental.pallas import tpu as pltpu
from jax.experimental.pallas import tpu_sc as plsc

D_MODEL = 1024
SEQ_LEN = 8192
NUM_CORES = 2
NUM_SUBCORES = 16
NUM_WORKERS = NUM_CORES * NUM_SUBCORES
ROWS_PER_WORKER = SEQ_LEN // NUM_WORKERS
CHUNK = 32
NUM_CHUNKS = ROWS_PER_WORKER // CHUNK


def _embed_body(idx_hbm, table_hbm, out_hbm, idx_v, buf0, buf1, sem0, sem1):
    wid = lax.axis_index("s") * NUM_CORES + lax.axis_index("c")
    pltpu.sync_copy(idx_hbm.at[wid], idx_v)
    bufs = (buf0, buf1)
    sems = (sem0, sem1)
    handles = [None, None]
    handles[0] = pltpu.async_copy(table_hbm.at[idx_v.at[0]], buf0, sem0)
    base = wid * ROWS_PER_WORKER
    for c in range(NUM_CHUNKS):
        b = c % 2
        if c + 1 < NUM_CHUNKS:
            nb = (c + 1) % 2
            handles[nb] = pltpu.async_copy(
                table_hbm.at[idx_v.at[c + 1]], bufs[nb], sems[nb])
        handles[b].wait()
        pltpu.sync_copy(bufs[b], out_hbm.at[pl.ds(base + c * CHUNK, CHUNK)])


_embed_gather = functools.partial(
    pl.kernel,
    mesh=plsc.VectorSubcoreMesh(core_axis_name="c", subcore_axis_name="s"),
    out_type=jax.ShapeDtypeStruct((SEQ_LEN, D_MODEL), jnp.float32),
    scratch_types=[
        pltpu.VMEM((NUM_CHUNKS, CHUNK), jnp.int32),
        pltpu.VMEM((CHUNK, D_MODEL), jnp.float32),
        pltpu.VMEM((CHUNK, D_MODEL), jnp.float32),
        pltpu.SemaphoreType.DMA,
        pltpu.SemaphoreType.DMA,
    ],
)(_embed_body)


def kernel(positions, weight):
    idx = positions.astype(jnp.int32).reshape(NUM_WORKERS, NUM_CHUNKS, CHUNK)
    return _embed_gather(idx, weight)


# speedup vs baseline: 1.5551x; 1.0159x over previous
import functools

import jax
import jax.numpy as jnp
from jax import lax
from jax.experimental import pallas as pl
from jax.experimental.pallas import tpu as pltpu
from jax.experimental.pallas import tpu_sc as plsc

D_MODEL = 1024
SEQ_LEN = 8192
NUM_CORES = 2
NUM_SUBCORES = 16
NUM_WORKERS = NUM_CORES * NUM_SUBCORES
ROWS_PER_WORKER = SEQ_LEN // NUM_WORKERS
CHUNK = 32
NUM_CHUNKS = ROWS_PER_WORKER // CHUNK


NBUF = 3


def _embed_body(idx_hbm, table_hbm, out_hbm, idx_v,
                buf0, buf1, buf2, gs0, gs1, gs2, ss0, ss1, ss2):
    wid = lax.axis_index("s") * NUM_CORES + lax.axis_index("c")
    pltpu.sync_copy(idx_hbm.at[wid], idx_v)
    bufs = (buf0, buf1, buf2)
    gsems = (gs0, gs1, gs2)
    ssems = (ss0, ss1, ss2)
    ghandles = [None] * NBUF
    shandles = [None] * NBUF
    base = wid * ROWS_PER_WORKER
    for c in range(min(NBUF - 1, NUM_CHUNKS)):
        b = c % NBUF
        ghandles[b] = pltpu.async_copy(table_hbm.at[idx_v.at[c]], bufs[b], gsems[b])
    for c in range(NUM_CHUNKS):
        b = c % NBUF
        nxt = c + NBUF - 1
        if nxt < NUM_CHUNKS:
            nb = nxt % NBUF
            if shandles[nb] is not None:
                shandles[nb].wait()
            ghandles[nb] = pltpu.async_copy(
                table_hbm.at[idx_v.at[nxt]], bufs[nb], gsems[nb])
        ghandles[b].wait()
        shandles[b] = pltpu.async_copy(
            bufs[b], out_hbm.at[pl.ds(base + c * CHUNK, CHUNK)], ssems[b])
    for c in range(max(0, NUM_CHUNKS - NBUF + 1), NUM_CHUNKS):
        shandles[c % NBUF].wait()


_embed_gather = functools.partial(
    pl.kernel,
    mesh=plsc.VectorSubcoreMesh(core_axis_name="c", subcore_axis_name="s"),
    out_type=jax.ShapeDtypeStruct((SEQ_LEN, D_MODEL), jnp.float32),
    scratch_types=[
        pltpu.VMEM((NUM_CHUNKS, CHUNK), jnp.int32),
        pltpu.VMEM((CHUNK, D_MODEL), jnp.float32),
        pltpu.VMEM((CHUNK, D_MODEL), jnp.float32),
        pltpu.VMEM((CHUNK, D_MODEL), jnp.float32),
        pltpu.SemaphoreType.DMA,
        pltpu.SemaphoreType.DMA,
        pltpu.SemaphoreType.DMA,
        pltpu.SemaphoreType.DMA,
        pltpu.SemaphoreType.DMA,
        pltpu.SemaphoreType.DMA,
    ],
)(_embed_body)


def kernel(positions, weight):
    idx = positions.astype(jnp.int32).reshape(NUM_WORKERS, NUM_CHUNKS, CHUNK)
    return _embed_gather(idx, weight)


# speedup vs baseline: 1.5815x; 1.0169x over previous
import functools

import jax
import jax.numpy as jnp
from jax import lax
from jax.experimental import pallas as pl
from jax.experimental.pallas import tpu as pltpu
from jax.experimental.pallas import tpu_sc as plsc

D_MODEL = 1024
SEQ_LEN = 8192
NUM_CORES = 2
NUM_SUBCORES = 16
NUM_WORKERS = NUM_CORES * NUM_SUBCORES
ROWS_PER_WORKER = SEQ_LEN // NUM_WORKERS
CHUNK = 16
NUM_CHUNKS = ROWS_PER_WORKER // CHUNK
NBUF = 7
LOOKAHEAD = 4


def _embed_body(idx_hbm, table_hbm, out_hbm, idx_v, *rest):
    bufs = rest[:NBUF]
    gsems = rest[NBUF:2 * NBUF]
    ssems = rest[2 * NBUF:]
    wid = lax.axis_index("s") * NUM_CORES + lax.axis_index("c")
    pltpu.sync_copy(idx_hbm.at[wid], idx_v)
    ghandles = [None] * NBUF
    shandles = [None] * NBUF
    base = wid * ROWS_PER_WORKER
    for c in range(min(LOOKAHEAD, NUM_CHUNKS)):
        b = c % NBUF
        ghandles[b] = pltpu.async_copy(table_hbm.at[idx_v.at[c]], bufs[b], gsems[b])
    for c in range(NUM_CHUNKS):
        b = c % NBUF
        g = c + LOOKAHEAD
        if g < NUM_CHUNKS:
            gb = g % NBUF
            if shandles[gb] is not None:
                shandles[gb].wait()
            ghandles[gb] = pltpu.async_copy(
                table_hbm.at[idx_v.at[g]], bufs[gb], gsems[gb])
        ghandles[b].wait()
        shandles[b] = pltpu.async_copy(
            bufs[b], out_hbm.at[pl.ds(base + c * CHUNK, CHUNK)], ssems[b])
    for c in range(max(0, NUM_CHUNKS - NBUF), NUM_CHUNKS):
        b = c % NBUF
        if shandles[b] is not None:
            shandles[b].wait()
            shandles[b] = None


_embed_gather = functools.partial(
    pl.kernel,
    mesh=plsc.VectorSubcoreMesh(core_axis_name="c", subcore_axis_name="s"),
    out_type=jax.ShapeDtypeStruct((SEQ_LEN, D_MODEL), jnp.float32),
    scratch_types=(
        [pltpu.VMEM((NUM_CHUNKS, CHUNK), jnp.int32)]
        + [pltpu.VMEM((CHUNK, D_MODEL), jnp.float32) for _ in range(NBUF)]
        + [pltpu.SemaphoreType.DMA for _ in range(2 * NBUF)]
    ),
)(_embed_body)


def kernel(positions, weight):
    idx = positions.astype(jnp.int32).reshape(NUM_WORKERS, NUM_CHUNKS, CHUNK)
    return _embed_gather(idx, weight)


# speedup vs baseline: 1.5834x; 1.0012x over previous
import functools

import jax
import jax.numpy as jnp
from jax import lax
from jax.experimental import pallas as pl
from jax.experimental.pallas import tpu as pltpu
from jax.experimental.pallas import tpu_sc as plsc

D_MODEL = 1024
SEQ_LEN = 8192
NUM_CORES = 2
NUM_SUBCORES = 16
NUM_WORKERS = NUM_CORES * NUM_SUBCORES
ROWS_PER_WORKER = SEQ_LEN // NUM_WORKERS
CHUNK = 16
NUM_CHUNKS = ROWS_PER_WORKER // CHUNK
NBUF = 7
LOOKAHEAD = 4


def _embed_body(idx_hbm, table_hbm, out_hbm, idx_v, *rest):
    bufs = rest[:NBUF]
    gsems = rest[NBUF:2 * NBUF]
    ssems = rest[2 * NBUF:]
    wid = lax.axis_index("s") * NUM_CORES + lax.axis_index("c")
    base = wid * ROWS_PER_WORKER
    pltpu.sync_copy(idx_hbm.at[pl.ds(base, ROWS_PER_WORKER)], idx_v)
    ghandles = [None] * NBUF
    shandles = [None] * NBUF
    for c in range(min(LOOKAHEAD, NUM_CHUNKS)):
        b = c % NBUF
        ghandles[b] = pltpu.async_copy(
            table_hbm.at[idx_v.at[pl.ds(c * CHUNK, CHUNK)]], bufs[b], gsems[b])
    for c in range(NUM_CHUNKS):
        b = c % NBUF
        g = c + LOOKAHEAD
        if g < NUM_CHUNKS:
            gb = g % NBUF
            if shandles[gb] is not None:
                shandles[gb].wait()
            ghandles[gb] = pltpu.async_copy(
                table_hbm.at[idx_v.at[pl.ds(g * CHUNK, CHUNK)]], bufs[gb], gsems[gb])
        ghandles[b].wait()
        shandles[b] = pltpu.async_copy(
            bufs[b], out_hbm.at[pl.ds(base + c * CHUNK, CHUNK)], ssems[b])
    for c in range(max(0, NUM_CHUNKS - NBUF), NUM_CHUNKS):
        b = c % NBUF
        if shandles[b] is not None:
            shandles[b].wait()
            shandles[b] = None


_embed_gather = functools.partial(
    pl.kernel,
    mesh=plsc.VectorSubcoreMesh(core_axis_name="c", subcore_axis_name="s"),
    out_type=jax.ShapeDtypeStruct((SEQ_LEN, D_MODEL), jnp.float32),
    scratch_types=(
        [pltpu.VMEM((ROWS_PER_WORKER,), jnp.int32)]
        + [pltpu.VMEM((CHUNK, D_MODEL), jnp.float32) for _ in range(NBUF)]
        + [pltpu.SemaphoreType.DMA for _ in range(2 * NBUF)]
    ),
)(_embed_body)


def kernel(positions, weight):
    return _embed_gather(positions.astype(jnp.int32), weight)
